# baseline (speedup 1.0000x reference)
import functools
import jax
import jax.numpy as jnp
import numpy as np
from jax import lax
from jax.experimental import pallas as pl
from jax.experimental.pallas import tpu as pltpu
from jax.experimental.pallas import tpu_sc as plsc

N = 4194304
TOPK = 4096
NC, NS = 2, 16
NW = NC * NS
L = 16

_SIGN = np.uint32(0x80000000)
_ZERO_U = np.uint32(0)


def _wid():
  return lax.axis_index("c") * NS + lax.axis_index("s")


def _zero2d(ref, rows, cols, dtype):
  z = jnp.zeros((L,), dtype)
  cl = cols // L

  def body(t, _):
    ref[t // cl, pl.ds((t % cl) * L, L)] = z
    return 0

  lax.fori_loop(0, rows * cl, body, 0, unroll=4)


def _key_from_len(ln):
  bits = plsc.bitcast(ln, jnp.uint32)
  biti = plsc.bitcast(ln, jnp.int32)
  u = jnp.where(biti < 0, ~bits, bits | _SIGN)
  return jnp.where(ln == 0.0, _ZERO_U, u)


def _sq_from_key(u):
  ui = plsc.bitcast(u, jnp.int32)
  bits = jnp.where(ui < 0, u ^ _SIGN, ~u)
  v = plsc.bitcast(bits, jnp.float32)
  return jnp.where(u == _ZERO_U, jnp.float32(0), v * v)


def _select_bin(cnt_ref, ssq_ref, nbins, cols, k_rem):
  def body(j, carry):
    tot, p_star, cnt_above, ssq_above = carry
    s = nbins - L * (j + 1)
    r = s // cols
    col = s % cols
    c = cnt_ref[0, r, pl.ds(col, L)] + cnt_ref[1, r, pl.ds(col, L)]
    q = ssq_ref[0, r, pl.ds(col, L)] + ssq_ref[1, r, pl.ds(col, L)]
    c_rev = lax.rev(c, (0,))
    q_rev = lax.rev(q, (0,))
    incl = tot + plsc.cumsum(c_rev)
    m = incl < k_rem
    p_star = p_star + jnp.sum(m.astype(jnp.int32))
    cnt_above = cnt_above + jnp.sum(jnp.where(m, c_rev, 0))
    ssq_above = ssq_above + jnp.sum(jnp.where(m, q_rev, jnp.float32(0)))
    tot = tot + jnp.sum(c_rev)
    return (tot, p_star, cnt_above, ssq_above)

  tot, p_star, cnt_above, ssq_above = lax.fori_loop(
      0, nbins // L, body, (jnp.int32(0), jnp.int32(0), jnp.int32(0),
                            jnp.float32(0)))
  b = (nbins - 1) - p_star
  return b, k_rem - cnt_above, ssq_above


def _iota():
  return lax.iota(jnp.int32, L)


def _reduce_tail(sid, cid, cnt_v, ssq_v, shc, shs, rbi, rbf, idx_v,
                 cnt_hbm, ssq_hbm):
  plsc.subcore_barrier()
  pltpu.sync_copy(cnt_v, shc.at[idx_v], add=True)
  pltpu.sync_copy(ssq_v, shs.at[idx_v], add=True)
  plsc.subcore_barrier()

  @pl.when(sid == 0)
  def _():
    pltpu.sync_copy(shc, rbi)
    pltpu.sync_copy(shs, rbf)
    pltpu.sync_copy(rbi, cnt_hbm.at[cid])
    pltpu.sync_copy(rbf, ssq_hbm.at[cid])


def _fill_iota(idx_v, rows):
  iota = _iota()
  for t in range(rows // L):
    idx_v[pl.ds(t * L, L)] = iota + t * L


def _make_k1(n, ch1, mesh, interpret):
  nt = n // NW
  nch = nt // ch1

  def body(dgm_hbm, u_hbm, cnt_hbm, ssq_hbm,
           dgm_v, u_v, cnt_v, ssq_v, rbi, rbf, idx_v, shc, shs):
    cid = lax.axis_index("c")
    sid = lax.axis_index("s")
    wid = _wid()
    iota = _iota()
    zeros = jnp.zeros((L,), jnp.int32)
    onesi = jnp.ones((L,), jnp.int32)
    ones = jnp.ones((L,), jnp.int32)

    _fill_iota(idx_v, 32)
    _zero2d(cnt_v, 32, 128, jnp.int32)
    _zero2d(ssq_v, 32, 128, jnp.float32)
    _zero2d(rbi, 32, 128, jnp.int32)
    _zero2d(rbf, 32, 128, jnp.float32)

    @pl.when(sid == 0)
    def _():
      pltpu.sync_copy(rbi, shc)
      pltpu.sync_copy(rbf, shs)

    base = wid * nt

    def chunk(c, _):
      pltpu.sync_copy(dgm_hbm.at[pl.ds(base + c * ch1, ch1)], dgm_v)

      def vec(i, _):
        rows = iota + i * L
        b = plsc.load_gather(dgm_v, [rows, zeros])
        d = plsc.load_gather(dgm_v, [rows, onesi])
        ln = d - b
        u = _key_from_len(ln)
        u_v[pl.ds(i * L, L)] = u
        r = (u >> np.uint32(27)).astype(jnp.int32)
        col = ((u >> np.uint32(20)) & np.uint32(127)).astype(jnp.int32)
        sq = ln * ln
        plsc.addupdate_scatter(cnt_v, [r, col], ones)
        plsc.addupdate_scatter(ssq_v, [r, col], sq)
        return 0

      lax.fori_loop(0, ch1 // L, vec, 0)
      pltpu.sync_copy(u_v, u_hbm.at[pl.ds(base + c * ch1, ch1)])
      return 0

    lax.fori_loop(0, nch, chunk, 0)
    _reduce_tail(sid, cid, cnt_v, ssq_v, shc, shs, rbi, rbf, idx_v,
                 cnt_hbm, ssq_hbm)

  return functools.partial(
      pl.kernel,
      out_type=(
          jax.ShapeDtypeStruct((n,), jnp.uint32),
          jax.ShapeDtypeStruct((NC, 32, 128), jnp.int32),
          jax.ShapeDtypeStruct((NC, 32, 128), jnp.float32),
      ),
      mesh=mesh,
      compiler_params=pltpu.CompilerParams(needs_layout_passes=False,
                                           use_tc_tiling_on_sc=False),
      scratch_types=[
          pltpu.VMEM((ch1, 2), jnp.float32),
          pltpu.VMEM((ch1,), jnp.uint32),
          pltpu.VMEM((32, 128), jnp.int32),
          pltpu.VMEM((32, 128), jnp.float32),
          pltpu.VMEM((32, 128), jnp.int32),
          pltpu.VMEM((32, 128), jnp.float32),
          pltpu.VMEM((32,), jnp.int32),
          pltpu.VMEM_SHARED((32, 128), jnp.int32),
          pltpu.VMEM_SHARED((32, 128), jnp.float32),
      ],
      interpret=interpret,
  )(body)


def _make_pass(level, n, ch2, mesh, interpret):
  nt = n // NW
  nch = nt // ch2
  rows, cols = (32, 128) if level == 2 else (16, 16)

  def body(u_hbm, c1_hbm, s1_hbm, c2_hbm, s2_hbm, cnt_out, ssq_out,
           u_v, h1c, h1s, h2c, h2s, cnt_v, ssq_v, rbi, rbf, idx_v,
           shc, shs):
    cid = lax.axis_index("c")
    sid = lax.axis_index("s")
    wid = _wid()
    ones = jnp.ones((L,), jnp.int32)

    pltpu.sync_copy(c1_hbm, h1c)
    pltpu.sync_copy(s1_hbm, h1s)
    if level == 3:
      pltpu.sync_copy(c2_hbm, h2c)
      pltpu.sync_copy(s2_hbm, h2s)

    b1, k1, _ = _select_bin(h1c, h1s, 4096, 128, np.int32(TOPK))
    if level == 2:
      prefix = b1.astype(jnp.uint32)
      shift = np.uint32(20)

      def bin_of(u):
        bb = (u >> np.uint32(8)) & np.uint32(0xFFF)
        return ((bb >> np.uint32(7)).astype(jnp.int32),
                (bb & np.uint32(127)).astype(jnp.int32))
    else:
      b2, _, _ = _select_bin(h2c, h2s, 4096, 128, k1)
      prefix = (b1.astype(jnp.uint32) << np.uint32(12)) | b2.astype(
          jnp.uint32)
      shift = np.uint32(8)

      def bin_of(u):
        bb = u & np.uint32(0xFF)
        return ((bb >> np.uint32(4)).astype(jnp.int32),
                (bb & np.uint32(15)).astype(jnp.int32))

    _fill_iota(idx_v, rows)
    _zero2d(cnt_v, rows, cols, jnp.int32)
    _zero2d(ssq_v, rows, cols, jnp.float32)
    _zero2d(rbi, rows, cols, jnp.int32)
    _zero2d(rbf, rows, cols, jnp.float32)

    @pl.when(sid == 0)
    def _():
      pltpu.sync_copy(rbi, shc)
      pltpu.sync_copy(rbf, shs)

    base = wid * nt

    def chunk(c, _):
      pltpu.sync_copy(u_hbm.at[pl.ds(base + c * ch2, ch2)], u_v)

      def vec(i, _):
        u = u_v[pl.ds(i * L, L)]
        match = (u >> shift) == prefix
        r, col = bin_of(u)
        sq = _sq_from_key(u)
        plsc.addupdate_scatter(cnt_v, [r, col], ones, mask=match)
        plsc.addupdate_scatter(ssq_v, [r, col], sq, mask=match)
        return 0

      lax.fori_loop(0, ch2 // L, vec, 0)
      return 0

    lax.fori_loop(0, nch, chunk, 0)
    _reduce_tail(sid, cid, cnt_v, ssq_v, shc, shs, rbi, rbf, idx_v,
                 cnt_out, ssq_out)

  return functools.partial(
      pl.kernel,
      out_type=(
          jax.ShapeDtypeStruct((NC, rows, cols), jnp.int32),
          jax.ShapeDtypeStruct((NC, rows, cols), jnp.float32),
      ),
      mesh=mesh,
      compiler_params=pltpu.CompilerParams(needs_layout_passes=False),
      scratch_types=[
          pltpu.VMEM((ch2,), jnp.uint32),
          pltpu.VMEM((NC, 32, 128), jnp.int32),
          pltpu.VMEM((NC, 32, 128), jnp.float32),
          pltpu.VMEM((NC, 32, 128), jnp.int32),
          pltpu.VMEM((NC, 32, 128), jnp.float32),
          pltpu.VMEM((rows, cols), jnp.int32),
          pltpu.VMEM((rows, cols), jnp.float32),
          pltpu.VMEM((rows, cols), jnp.int32),
          pltpu.VMEM((rows, cols), jnp.float32),
          pltpu.VMEM((rows,), jnp.int32),
          pltpu.VMEM_SHARED((rows, cols), jnp.int32),
          pltpu.VMEM_SHARED((rows, cols), jnp.float32),
      ],
      interpret=interpret,
  )(body)


def _make_k4(mesh, interpret):
  def body(c1, s1, c2, s2, c3, s3, out_hbm,
           h1c, h1s, h2c, h2s, h3c, h3s, out_v):
    wid = _wid()

    @pl.when(wid == 0)
    def _():
      pltpu.sync_copy(c1, h1c)
      pltpu.sync_copy(s1, h1s)
      pltpu.sync_copy(c2, h2c)
      pltpu.sync_copy(s2, h2s)
      pltpu.sync_copy(c3, h3c)
      pltpu.sync_copy(s3, h3s)
      b1, k1, ssq1 = _select_bin(h1c, h1s, 4096, 128, np.int32(TOPK))
      b2, k2, ssq2 = _select_bin(h2c, h2s, 4096, 128, k1)
      b3, k3, ssq3 = _select_bin(h3c, h3s, 256, 16, k2)
      ustar = ((b1.astype(jnp.uint32) << np.uint32(20))
               | (b2.astype(jnp.uint32) << np.uint32(8))
               | b3.astype(jnp.uint32))
      ustar_vec = jnp.zeros((L,), jnp.uint32) + ustar
      tsq = _sq_from_key(ustar_vec)
      loss = ssq1 + ssq2 + ssq3 + k3.astype(jnp.float32) * tsq
      out_v[...] = jnp.where(_iota() == 0, loss, jnp.float32(0))
      pltpu.sync_copy(out_v, out_hbm)

  return functools.partial(
      pl.kernel,
      out_type=jax.ShapeDtypeStruct((L,), jnp.float32),
      mesh=mesh,
      compiler_params=pltpu.CompilerParams(needs_layout_passes=False),
      scratch_types=[
          pltpu.VMEM((NC, 32, 128), jnp.int32),
          pltpu.VMEM((NC, 32, 128), jnp.float32),
          pltpu.VMEM((NC, 32, 128), jnp.int32),
          pltpu.VMEM((NC, 32, 128), jnp.float32),
          pltpu.VMEM((NC, 16, 16), jnp.int32),
          pltpu.VMEM((NC, 16, 16), jnp.float32),
          pltpu.VMEM((L,), jnp.float32),
      ],
      interpret=interpret,
  )(body)


def _build(n, ch1, ch2, interpret=False):
  mesh = plsc.VectorSubcoreMesh(core_axis_name="c", subcore_axis_name="s",
                                num_cores=NC, num_subcores=NS)
  k1 = _make_k1(n, ch1, mesh, interpret)
  k2 = _make_pass(2, n, ch2, mesh, interpret)
  k3 = _make_pass(3, n, ch2, mesh, interpret)
  k4 = _make_k4(mesh, interpret)

  def run(dgm):
    u, c1, s1 = k1(dgm)
    c2, s2 = k2(u, c1, s1, c1, s1)
    c3, s3 = k3(u, c1, s1, c2, s2)
    out = k4(c1, s1, c2, s2, c3, s3)
    return out[0]

  return run


_run = _build(N, 8192, 16384)


@jax.jit
def kernel(dgm):
  return _run(dgm)


# speedup vs baseline: 19.9216x; 19.9216x over previous
import functools
import jax
import jax.numpy as jnp
import numpy as np
from jax import lax
from jax.experimental import pallas as pl
from jax.experimental.pallas import tpu as pltpu
from jax.experimental.pallas import tpu_sc as plsc

N = 4194304
TOPK = 4096
NC, NS = 2, 16
NW = NC * NS
L = 16

_SIGN = np.uint32(0x80000000)
_ZERO_U = np.uint32(0)


def _wid():
  return lax.axis_index("c") * NS + lax.axis_index("s")


def _zero2d(ref, rows, cols, dtype):
  z = jnp.zeros((L,), dtype)
  cl = cols // L

  def body(t, _):
    ref[t // cl, pl.ds((t % cl) * L, L)] = z
    return 0

  lax.fori_loop(0, rows * cl, body, 0, unroll=4)


def _key_from_len(ln):
  bits = plsc.bitcast(ln, jnp.uint32)
  biti = plsc.bitcast(ln, jnp.int32)
  u = jnp.where(biti < 0, ~bits, bits | _SIGN)
  return jnp.where(ln == 0.0, _ZERO_U, u)


def _sq_from_key(u):
  ui = plsc.bitcast(u, jnp.int32)
  bits = jnp.where(ui < 0, u ^ _SIGN, ~u)
  v = plsc.bitcast(bits, jnp.float32)
  return jnp.where(u == _ZERO_U, jnp.float32(0), v * v)


def _select_bin(cnt_ref, ssq_ref, nbins, cols, k_rem):
  def body(j, carry):
    tot, p_star, cnt_above, ssq_above = carry
    s = nbins - L * (j + 1)
    r = s // cols
    col = s % cols
    c = cnt_ref[0, r, pl.ds(col, L)] + cnt_ref[1, r, pl.ds(col, L)]
    q = ssq_ref[0, r, pl.ds(col, L)] + ssq_ref[1, r, pl.ds(col, L)]
    c_rev = lax.rev(c, (0,))
    q_rev = lax.rev(q, (0,))
    incl = tot + plsc.cumsum(c_rev)
    m = incl < k_rem
    p_star = p_star + jnp.sum(m.astype(jnp.int32))
    cnt_above = cnt_above + jnp.sum(jnp.where(m, c_rev, 0))
    ssq_above = ssq_above + jnp.sum(jnp.where(m, q_rev, jnp.float32(0)))
    tot = tot + jnp.sum(c_rev)
    return (tot, p_star, cnt_above, ssq_above)

  tot, p_star, cnt_above, ssq_above = lax.fori_loop(
      0, nbins // L, body, (jnp.int32(0), jnp.int32(0), jnp.int32(0),
                            jnp.float32(0)))
  b = (nbins - 1) - p_star
  return b, k_rem - cnt_above, ssq_above


def _iota():
  return lax.iota(jnp.int32, L)


def _reduce_tail(sid, cid, cnt_v, ssq_v, shc, shs, rbi, rbf, idx_v,
                 cnt_hbm, ssq_hbm):
  plsc.subcore_barrier()
  pltpu.sync_copy(cnt_v, shc.at[idx_v], add=True)
  pltpu.sync_copy(ssq_v, shs.at[idx_v], add=True)
  plsc.subcore_barrier()

  @pl.when(sid == 0)
  def _():
    pltpu.sync_copy(shc, rbi)
    pltpu.sync_copy(shs, rbf)
    pltpu.sync_copy(rbi, cnt_hbm.at[cid])
    pltpu.sync_copy(rbf, ssq_hbm.at[cid])


def _fill_iota(idx_v, rows):
  iota = _iota()
  for t in range(rows // L):
    idx_v[pl.ds(t * L, L)] = iota + t * L


def _make_k1(n, ch1, mesh, interpret):
  nt = n // NW
  nch = nt // ch1

  def body(dgm_hbm, u_hbm, cnt_hbm, ssq_hbm,
           dgm_v, u_v, cnt_v, ssq_v, rbi, rbf, idx_v, shc, shs):
    cid = lax.axis_index("c")
    sid = lax.axis_index("s")
    wid = _wid()
    ones = jnp.ones((L,), jnp.int32)

    _fill_iota(idx_v, 32)
    _zero2d(cnt_v, 32, 128, jnp.int32)
    _zero2d(ssq_v, 32, 128, jnp.float32)
    _zero2d(rbi, 32, 128, jnp.int32)
    _zero2d(rbf, 32, 128, jnp.float32)

    @pl.when(sid == 0)
    def _():
      pltpu.sync_copy(rbi, shc)
      pltpu.sync_copy(rbf, shs)

    base = wid * nt
    cb = ch1 // 128
    base_blk = base // 128

    def chunk(c, _):
      pltpu.sync_copy(dgm_hbm.at[pl.ds(base_blk + c * cb, cb)], dgm_v)

      def vec(i, _):
        blk = i // 8
        v = (i % 8) * L
        b = dgm_v[blk, 0, pl.ds(v, L)]
        d = dgm_v[blk, 1, pl.ds(v, L)]
        ln = d - b
        u = _key_from_len(ln)
        u_v[pl.ds(i * L, L)] = u
        r = (u >> np.uint32(27)).astype(jnp.int32)
        col = ((u >> np.uint32(20)) & np.uint32(127)).astype(jnp.int32)
        sq = ln * ln
        plsc.addupdate_scatter(cnt_v, [r, col], ones)
        plsc.addupdate_scatter(ssq_v, [r, col], sq)
        return 0

      lax.fori_loop(0, ch1 // L, vec, 0)
      pltpu.sync_copy(u_v, u_hbm.at[pl.ds(base + c * ch1, ch1)])
      return 0

    lax.fori_loop(0, nch, chunk, 0)
    _reduce_tail(sid, cid, cnt_v, ssq_v, shc, shs, rbi, rbf, idx_v,
                 cnt_hbm, ssq_hbm)

  return functools.partial(
      pl.kernel,
      out_type=(
          jax.ShapeDtypeStruct((n,), jnp.uint32),
          jax.ShapeDtypeStruct((NC, 32, 128), jnp.int32),
          jax.ShapeDtypeStruct((NC, 32, 128), jnp.float32),
      ),
      mesh=mesh,
      compiler_params=pltpu.CompilerParams(needs_layout_passes=False,
                                           use_tc_tiling_on_sc=False),
      scratch_types=[
          pltpu.VMEM((ch1 // 128, 2, 128), jnp.float32),
          pltpu.VMEM((ch1,), jnp.uint32),
          pltpu.VMEM((32, 128), jnp.int32),
          pltpu.VMEM((32, 128), jnp.float32),
          pltpu.VMEM((32, 128), jnp.int32),
          pltpu.VMEM((32, 128), jnp.float32),
          pltpu.VMEM((32,), jnp.int32),
          pltpu.VMEM_SHARED((32, 128), jnp.int32),
          pltpu.VMEM_SHARED((32, 128), jnp.float32),
      ],
      interpret=interpret,
  )(body)


def _make_pass(level, n, ch2, mesh, interpret):
  nt = n // NW
  nch = nt // ch2
  rows, cols = (32, 128) if level == 2 else (16, 16)

  def body(u_hbm, c1_hbm, s1_hbm, c2_hbm, s2_hbm, cnt_out, ssq_out,
           u_v, h1c, h1s, h2c, h2s, cnt_v, ssq_v, rbi, rbf, idx_v,
           shc, shs):
    cid = lax.axis_index("c")
    sid = lax.axis_index("s")
    wid = _wid()
    ones = jnp.ones((L,), jnp.int32)

    pltpu.sync_copy(c1_hbm, h1c)
    pltpu.sync_copy(s1_hbm, h1s)
    if level == 3:
      pltpu.sync_copy(c2_hbm, h2c)
      pltpu.sync_copy(s2_hbm, h2s)

    b1, k1, _ = _select_bin(h1c, h1s, 4096, 128, np.int32(TOPK))
    if level == 2:
      prefix = b1.astype(jnp.uint32)
      shift = np.uint32(20)

      def bin_of(u):
        bb = (u >> np.uint32(8)) & np.uint32(0xFFF)
        return ((bb >> np.uint32(7)).astype(jnp.int32),
                (bb & np.uint32(127)).astype(jnp.int32))
    else:
      b2, _, _ = _select_bin(h2c, h2s, 4096, 128, k1)
      prefix = (b1.astype(jnp.uint32) << np.uint32(12)) | b2.astype(
          jnp.uint32)
      shift = np.uint32(8)

      def bin_of(u):
        bb = u & np.uint32(0xFF)
        return ((bb >> np.uint32(4)).astype(jnp.int32),
                (bb & np.uint32(15)).astype(jnp.int32))

    _fill_iota(idx_v, rows)
    _zero2d(cnt_v, rows, cols, jnp.int32)
    _zero2d(ssq_v, rows, cols, jnp.float32)
    _zero2d(rbi, rows, cols, jnp.int32)
    _zero2d(rbf, rows, cols, jnp.float32)

    @pl.when(sid == 0)
    def _():
      pltpu.sync_copy(rbi, shc)
      pltpu.sync_copy(rbf, shs)

    base = wid * nt

    def chunk(c, _):
      pltpu.sync_copy(u_hbm.at[pl.ds(base + c * ch2, ch2)], u_v)

      def vec(i, _):
        u = u_v[pl.ds(i * L, L)]
        match = (u >> shift) == prefix
        r, col = bin_of(u)
        sq = _sq_from_key(u)
        plsc.addupdate_scatter(cnt_v, [r, col], ones, mask=match)
        plsc.addupdate_scatter(ssq_v, [r, col], sq, mask=match)
        return 0

      lax.fori_loop(0, ch2 // L, vec, 0)
      return 0

    lax.fori_loop(0, nch, chunk, 0)
    _reduce_tail(sid, cid, cnt_v, ssq_v, shc, shs, rbi, rbf, idx_v,
                 cnt_out, ssq_out)

  return functools.partial(
      pl.kernel,
      out_type=(
          jax.ShapeDtypeStruct((NC, rows, cols), jnp.int32),
          jax.ShapeDtypeStruct((NC, rows, cols), jnp.float32),
      ),
      mesh=mesh,
      compiler_params=pltpu.CompilerParams(needs_layout_passes=False),
      scratch_types=[
          pltpu.VMEM((ch2,), jnp.uint32),
          pltpu.VMEM((NC, 32, 128), jnp.int32),
          pltpu.VMEM((NC, 32, 128), jnp.float32),
          pltpu.VMEM((NC, 32, 128), jnp.int32),
          pltpu.VMEM((NC, 32, 128), jnp.float32),
          pltpu.VMEM((rows, cols), jnp.int32),
          pltpu.VMEM((rows, cols), jnp.float32),
          pltpu.VMEM((rows, cols), jnp.int32),
          pltpu.VMEM((rows, cols), jnp.float32),
          pltpu.VMEM((rows,), jnp.int32),
          pltpu.VMEM_SHARED((rows, cols), jnp.int32),
          pltpu.VMEM_SHARED((rows, cols), jnp.float32),
      ],
      interpret=interpret,
  )(body)


def _make_k4(mesh, interpret):
  def body(c1, s1, c2, s2, c3, s3, out_hbm,
           h1c, h1s, h2c, h2s, h3c, h3s, out_v):
    wid = _wid()

    @pl.when(wid == 0)
    def _():
      pltpu.sync_copy(c1, h1c)
      pltpu.sync_copy(s1, h1s)
      pltpu.sync_copy(c2, h2c)
      pltpu.sync_copy(s2, h2s)
      pltpu.sync_copy(c3, h3c)
      pltpu.sync_copy(s3, h3s)
      b1, k1, ssq1 = _select_bin(h1c, h1s, 4096, 128, np.int32(TOPK))
      b2, k2, ssq2 = _select_bin(h2c, h2s, 4096, 128, k1)
      b3, k3, ssq3 = _select_bin(h3c, h3s, 256, 16, k2)
      ustar = ((b1.astype(jnp.uint32) << np.uint32(20))
               | (b2.astype(jnp.uint32) << np.uint32(8))
               | b3.astype(jnp.uint32))
      ustar_vec = jnp.zeros((L,), jnp.uint32) + ustar
      tsq = _sq_from_key(ustar_vec)
      loss = ssq1 + ssq2 + ssq3 + k3.astype(jnp.float32) * tsq
      out_v[...] = jnp.where(_iota() == 0, loss, jnp.float32(0))
      pltpu.sync_copy(out_v, out_hbm)

  return functools.partial(
      pl.kernel,
      out_type=jax.ShapeDtypeStruct((L,), jnp.float32),
      mesh=mesh,
      compiler_params=pltpu.CompilerParams(needs_layout_passes=False),
      scratch_types=[
          pltpu.VMEM((NC, 32, 128), jnp.int32),
          pltpu.VMEM((NC, 32, 128), jnp.float32),
          pltpu.VMEM((NC, 32, 128), jnp.int32),
          pltpu.VMEM((NC, 32, 128), jnp.float32),
          pltpu.VMEM((NC, 16, 16), jnp.int32),
          pltpu.VMEM((NC, 16, 16), jnp.float32),
          pltpu.VMEM((L,), jnp.float32),
      ],
      interpret=interpret,
  )(body)


def _build(n, ch1, ch2, interpret=False):
  mesh = plsc.VectorSubcoreMesh(core_axis_name="c", subcore_axis_name="s",
                                num_cores=NC, num_subcores=NS)
  k1 = _make_k1(n, ch1, mesh, interpret)
  k2 = _make_pass(2, n, ch2, mesh, interpret)
  k3 = _make_pass(3, n, ch2, mesh, interpret)
  k4 = _make_k4(mesh, interpret)

  def run(dgm):
    dgm_b = dgm.reshape(-1, 128, 2).transpose(0, 2, 1)
    u, c1, s1 = k1(dgm_b)
    c2, s2 = k2(u, c1, s1, c1, s1)
    c3, s3 = k3(u, c1, s1, c2, s2)
    out = k4(c1, s1, c2, s2, c3, s3)
    return out[0]

  return run


_run = _build(N, 8192, 16384)


@jax.jit
def kernel(dgm):
  return _run(dgm)


# speedup vs baseline: 30.9219x; 1.5522x over previous
import functools
import jax
import jax.numpy as jnp
import numpy as np
from jax import lax
from jax.experimental import pallas as pl
from jax.experimental.pallas import tpu as pltpu
from jax.experimental.pallas import tpu_sc as plsc

N = 4194304
TOPK = 4096
NC, NS = 2, 16
NW = NC * NS
L = 16

_SIGN = np.uint32(0x80000000)
_ZERO_U = np.uint32(0)


def _wid():
  return lax.axis_index("c") * NS + lax.axis_index("s")


def _zero2d(ref, rows, cols, dtype):
  z = jnp.zeros((L,), dtype)
  cl = cols // L

  def body(t, _):
    ref[t // cl, pl.ds((t % cl) * L, L)] = z
    return 0

  lax.fori_loop(0, rows * cl, body, 0, unroll=4)


def _key_from_len(ln):
  bits = plsc.bitcast(ln, jnp.uint32)
  biti = plsc.bitcast(ln, jnp.int32)
  u = jnp.where(biti < 0, ~bits, bits | _SIGN)
  return jnp.where(ln == 0.0, _ZERO_U, u)


def _sq_from_key(u):
  ui = plsc.bitcast(u, jnp.int32)
  bits = jnp.where(ui < 0, u ^ _SIGN, ~u)
  v = plsc.bitcast(bits, jnp.float32)
  return jnp.where(u == _ZERO_U, jnp.float32(0), v * v)


def _select_bin(cnt_ref, ssq_ref, nbins, cols, k_rem):
  def body(j, carry):
    tot, p_star, cnt_above, ssq_above = carry
    s = nbins - L * (j + 1)
    r = s // cols
    col = s % cols
    c = cnt_ref[0, r, pl.ds(col, L)] + cnt_ref[1, r, pl.ds(col, L)]
    q = ssq_ref[0, r, pl.ds(col, L)] + ssq_ref[1, r, pl.ds(col, L)]
    c_rev = lax.rev(c, (0,))
    q_rev = lax.rev(q, (0,))
    incl = tot + plsc.cumsum(c_rev)
    m = incl < k_rem
    p_star = p_star + jnp.sum(m.astype(jnp.int32))
    cnt_above = cnt_above + jnp.sum(jnp.where(m, c_rev, 0))
    ssq_above = ssq_above + jnp.sum(jnp.where(m, q_rev, jnp.float32(0)))
    tot = tot + jnp.sum(c_rev)
    return (tot, p_star, cnt_above, ssq_above)

  tot, p_star, cnt_above, ssq_above = lax.fori_loop(
      0, nbins // L, body, (jnp.int32(0), jnp.int32(0), jnp.int32(0),
                            jnp.float32(0)))
  b = (nbins - 1) - p_star
  return b, k_rem - cnt_above, ssq_above


def _iota():
  return lax.iota(jnp.int32, L)


def _reduce_tail(sid, cid, cnt_v, ssq_v, shc, shs, rbi, rbf, idx_v,
                 cnt_hbm, ssq_hbm):
  plsc.subcore_barrier()
  pltpu.sync_copy(cnt_v, shc.at[idx_v], add=True)
  pltpu.sync_copy(ssq_v, shs.at[idx_v], add=True)
  plsc.subcore_barrier()

  @pl.when(sid == 0)
  def _():
    pltpu.sync_copy(shc, rbi)
    pltpu.sync_copy(shs, rbf)
    pltpu.sync_copy(rbi, cnt_hbm.at[cid])
    pltpu.sync_copy(rbf, ssq_hbm.at[cid])


def _fill_iota(idx_v, rows):
  iota = _iota()
  for t in range(rows // L):
    idx_v[pl.ds(t * L, L)] = iota + t * L


_params = pltpu.CompilerParams(needs_layout_passes=False,
                               use_tc_tiling_on_sc=False)


def _make_k1(n, ch1, mesh, interpret):
  nt = n // NW
  nch = nt // ch1
  cb = ch1 // 128
  assert nch % 2 == 0

  def body(dgm_hbm, u_hbm, cnt_hbm, ssq_hbm,
           dgm_v0, dgm_v1, u_v0, u_v1, cnt_v, ssq_v, rbi, rbf, idx_v,
           shc, shs, dsa, dsb, usa, usb):
    cid = lax.axis_index("c")
    sid = lax.axis_index("s")
    wid = _wid()
    ones = jnp.ones((L,), jnp.int32)

    _fill_iota(idx_v, 32)
    _zero2d(cnt_v, 32, 128, jnp.int32)
    _zero2d(ssq_v, 32, 128, jnp.float32)
    _zero2d(rbi, 32, 128, jnp.int32)
    _zero2d(rbf, 32, 128, jnp.float32)

    @pl.when(sid == 0)
    def _():
      pltpu.sync_copy(rbi, shc)
      pltpu.sync_copy(rbf, shs)

    base = wid * nt
    base_blk = base // 128
    bufs = [(dgm_v0, dsa, u_v0, usa),
            (dgm_v1, dsb, u_v1, usb)]

    pltpu.async_copy(dgm_hbm.at[pl.ds(base_blk, cb)], dgm_v0, dsa)

    def pair(c2, _):
      for p in range(2):
        buf, dsem, ubuf, usem = bufs[p]
        nbuf, ndsem = bufs[1 - p][0], bufs[1 - p][1]
        c = c2 * 2 + p
        pltpu.make_async_copy(dgm_hbm.at[pl.ds(0, cb)], buf, dsem).wait()

        @pl.when(c + 1 < nch)
        def _():
          pltpu.async_copy(dgm_hbm.at[pl.ds(base_blk + (c + 1) * cb, cb)],
                           nbuf, ndsem)

        @pl.when(c2 >= 1)
        def _():
          pltpu.make_async_copy(ubuf, u_hbm.at[pl.ds(0, ch1)], usem).wait()

        def blkbody(blk, _):
          for v in range(8):
            b = buf[blk, 0, pl.ds(v * L, L)]
            d = buf[blk, 1, pl.ds(v * L, L)]
            ln = d - b
            u = _key_from_len(ln)
            ubuf[pl.ds(blk * 128 + v * L, L)] = u
            r = (u >> np.uint32(27)).astype(jnp.int32)
            col = ((u >> np.uint32(20)) & np.uint32(127)).astype(jnp.int32)
            sq = ln * ln
            plsc.addupdate_scatter(cnt_v, [r, col], ones)
            plsc.addupdate_scatter(ssq_v, [r, col], sq)
          return 0

        lax.fori_loop(0, cb, blkbody, 0)
        pltpu.async_copy(ubuf, u_hbm.at[pl.ds(base + c * ch1, ch1)], usem)
      return 0

    lax.fori_loop(0, nch // 2, pair, 0)
    pltpu.make_async_copy(u_v0, u_hbm.at[pl.ds(0, ch1)], usa).wait()
    pltpu.make_async_copy(u_v1, u_hbm.at[pl.ds(0, ch1)], usb).wait()
    _reduce_tail(sid, cid, cnt_v, ssq_v, shc, shs, rbi, rbf, idx_v,
                 cnt_hbm, ssq_hbm)

  return functools.partial(
      pl.kernel,
      out_type=(
          jax.ShapeDtypeStruct((n,), jnp.uint32),
          jax.ShapeDtypeStruct((NC, 32, 128), jnp.int32),
          jax.ShapeDtypeStruct((NC, 32, 128), jnp.float32),
      ),
      mesh=mesh,
      compiler_params=_params,
      scratch_types=[
          pltpu.VMEM((ch1 // 128, 2, 128), jnp.float32),
          pltpu.VMEM((ch1 // 128, 2, 128), jnp.float32),
          pltpu.VMEM((ch1,), jnp.uint32),
          pltpu.VMEM((ch1,), jnp.uint32),
          pltpu.VMEM((32, 128), jnp.int32),
          pltpu.VMEM((32, 128), jnp.float32),
          pltpu.VMEM((32, 128), jnp.int32),
          pltpu.VMEM((32, 128), jnp.float32),
          pltpu.VMEM((32,), jnp.int32),
          pltpu.VMEM_SHARED((32, 128), jnp.int32),
          pltpu.VMEM_SHARED((32, 128), jnp.float32),
          pltpu.SemaphoreType.DMA,
          pltpu.SemaphoreType.DMA,
          pltpu.SemaphoreType.DMA,
          pltpu.SemaphoreType.DMA,
      ],
      interpret=interpret,
  )(body)


def _make_pass(level, n, ch2, mesh, interpret):
  nt = n // NW
  nch = nt // ch2
  rows, cols = (32, 128) if level == 2 else (16, 16)
  assert nch % 2 == 0

  def body(u_hbm, c1_hbm, s1_hbm, c2_hbm, s2_hbm, cnt_out, ssq_out,
           u_v0, u_v1, h1c, h1s, h2c, h2s, cnt_v, ssq_v, rbi, rbf, idx_v,
           shc, shs, dsa, dsb):
    cid = lax.axis_index("c")
    sid = lax.axis_index("s")
    wid = _wid()
    ones = jnp.ones((L,), jnp.int32)

    pltpu.sync_copy(c1_hbm, h1c)
    pltpu.sync_copy(s1_hbm, h1s)
    if level == 3:
      pltpu.sync_copy(c2_hbm, h2c)
      pltpu.sync_copy(s2_hbm, h2s)

    b1, k1, _ = _select_bin(h1c, h1s, 4096, 128, np.int32(TOPK))
    if level == 2:
      prefix = b1.astype(jnp.uint32)
      shift = np.uint32(20)

      def bin_of(u):
        bb = (u >> np.uint32(8)) & np.uint32(0xFFF)
        return ((bb >> np.uint32(7)).astype(jnp.int32),
                (bb & np.uint32(127)).astype(jnp.int32))
    else:
      b2, _, _ = _select_bin(h2c, h2s, 4096, 128, k1)
      prefix = (b1.astype(jnp.uint32) << np.uint32(12)) | b2.astype(
          jnp.uint32)
      shift = np.uint32(8)

      def bin_of(u):
        bb = u & np.uint32(0xFF)
        return ((bb >> np.uint32(4)).astype(jnp.int32),
                (bb & np.uint32(15)).astype(jnp.int32))

    _fill_iota(idx_v, rows)
    _zero2d(cnt_v, rows, cols, jnp.int32)
    _zero2d(ssq_v, rows, cols, jnp.float32)
    _zero2d(rbi, rows, cols, jnp.int32)
    _zero2d(rbf, rows, cols, jnp.float32)

    @pl.when(sid == 0)
    def _():
      pltpu.sync_copy(rbi, shc)
      pltpu.sync_copy(rbf, shs)

    base = wid * nt
    bufs = [(u_v0, dsa), (u_v1, dsb)]

    pltpu.async_copy(u_hbm.at[pl.ds(base, ch2)], u_v0, dsa)

    def pair(c2_, _):
      for p in range(2):
        buf, dsem = bufs[p]
        nbuf, ndsem = bufs[1 - p]
        c = c2_ * 2 + p
        pltpu.make_async_copy(u_hbm.at[pl.ds(0, ch2)], buf, dsem).wait()

        @pl.when(c + 1 < nch)
        def _():
          pltpu.async_copy(u_hbm.at[pl.ds(base + (c + 1) * ch2, ch2)],
                           nbuf, ndsem)

        def vec8(g, _):
          us = []
          ms = []
          for t in range(8):
            u = buf[pl.ds(g * 128 + t * L, L)]
            us.append(u)
            ms.append((u >> shift) == prefix)
          anym = ms[0]
          for t in range(1, 8):
            anym = anym | ms[t]

          @pl.when(jnp.any(anym))
          def _():
            for t in range(8):
              u = us[t]
              r, col = bin_of(u)
              sq = _sq_from_key(u)
              plsc.addupdate_scatter(cnt_v, [r, col], ones, mask=ms[t])
              plsc.addupdate_scatter(ssq_v, [r, col], sq, mask=ms[t])

          return 0

        lax.fori_loop(0, ch2 // 128, vec8, 0)
      return 0

    lax.fori_loop(0, nch // 2, pair, 0)
    _reduce_tail(sid, cid, cnt_v, ssq_v, shc, shs, rbi, rbf, idx_v,
                 cnt_out, ssq_out)

  return functools.partial(
      pl.kernel,
      out_type=(
          jax.ShapeDtypeStruct((NC, rows, cols), jnp.int32),
          jax.ShapeDtypeStruct((NC, rows, cols), jnp.float32),
      ),
      mesh=mesh,
      compiler_params=_params,
      scratch_types=[
          pltpu.VMEM((ch2,), jnp.uint32),
          pltpu.VMEM((ch2,), jnp.uint32),
          pltpu.VMEM((NC, 32, 128), jnp.int32),
          pltpu.VMEM((NC, 32, 128), jnp.float32),
          pltpu.VMEM((NC, 32, 128), jnp.int32),
          pltpu.VMEM((NC, 32, 128), jnp.float32),
          pltpu.VMEM((rows, cols), jnp.int32),
          pltpu.VMEM((rows, cols), jnp.float32),
          pltpu.VMEM((rows, cols), jnp.int32),
          pltpu.VMEM((rows, cols), jnp.float32),
          pltpu.VMEM((rows,), jnp.int32),
          pltpu.VMEM_SHARED((rows, cols), jnp.int32),
          pltpu.VMEM_SHARED((rows, cols), jnp.float32),
          pltpu.SemaphoreType.DMA,
          pltpu.SemaphoreType.DMA,
      ],
      interpret=interpret,
  )(body)


def _make_k4(mesh, interpret):
  def body(c1, s1, c2, s2, c3, s3, out_hbm,
           h1c, h1s, h2c, h2s, h3c, h3s, out_v):
    wid = _wid()

    @pl.when(wid == 0)
    def _():
      pltpu.sync_copy(c1, h1c)
      pltpu.sync_copy(s1, h1s)
      pltpu.sync_copy(c2, h2c)
      pltpu.sync_copy(s2, h2s)
      pltpu.sync_copy(c3, h3c)
      pltpu.sync_copy(s3, h3s)
      b1, k1, ssq1 = _select_bin(h1c, h1s, 4096, 128, np.int32(TOPK))
      b2, k2, ssq2 = _select_bin(h2c, h2s, 4096, 128, k1)
      b3, k3, ssq3 = _select_bin(h3c, h3s, 256, 16, k2)
      ustar = ((b1.astype(jnp.uint32) << np.uint32(20))
               | (b2.astype(jnp.uint32) << np.uint32(8))
               | b3.astype(jnp.uint32))
      ustar_vec = jnp.zeros((L,), jnp.uint32) + ustar
      tsq = _sq_from_key(ustar_vec)
      loss = ssq1 + ssq2 + ssq3 + k3.astype(jnp.float32) * tsq
      out_v[...] = jnp.where(_iota() == 0, loss, jnp.float32(0))
      pltpu.sync_copy(out_v, out_hbm)

  return functools.partial(
      pl.kernel,
      out_type=jax.ShapeDtypeStruct((L,), jnp.float32),
      mesh=mesh,
      compiler_params=_params,
      scratch_types=[
          pltpu.VMEM((NC, 32, 128), jnp.int32),
          pltpu.VMEM((NC, 32, 128), jnp.float32),
          pltpu.VMEM((NC, 32, 128), jnp.int32),
          pltpu.VMEM((NC, 32, 128), jnp.float32),
          pltpu.VMEM((NC, 16, 16), jnp.int32),
          pltpu.VMEM((NC, 16, 16), jnp.float32),
          pltpu.VMEM((L,), jnp.float32),
      ],
      interpret=interpret,
  )(body)


def _build(n, ch1, ch2, interpret=False):
  mesh = plsc.VectorSubcoreMesh(core_axis_name="c", subcore_axis_name="s",
                                num_cores=NC, num_subcores=NS)
  k1 = _make_k1(n, ch1, mesh, interpret)
  k2 = _make_pass(2, n, ch2, mesh, interpret)
  k3 = _make_pass(3, n, ch2, mesh, interpret)
  k4 = _make_k4(mesh, interpret)

  def run(dgm):
    dgm_b = dgm.reshape(-1, 128, 2).transpose(0, 2, 1)
    u, c1, s1 = k1(dgm_b)
    c2, s2 = k2(u, c1, s1, c1, s1)
    c3, s3 = k3(u, c1, s1, c2, s2)
    out = k4(c1, s1, c2, s2, c3, s3)
    return out[0]

  return run


_run = _build(N, 8192, 16384)


@jax.jit
def kernel(dgm):
  return _run(dgm)
